# Initial kernel scaffold; baseline (speedup 1.0000x reference)
import jax
import jax.numpy as jnp
from jax.experimental import pallas as pl


def kernel(efeat, nfeat, edge_index, w1, b1, w2, b2, ln_scale, ln_bias):
    raise NotImplementedError("write your pallas kernel here")


# speedup vs baseline: 1.5707x; 1.5707x over previous
import functools

import jax
import jax.numpy as jnp
from jax import lax
from jax.experimental import pallas as pl
from jax.experimental.pallas import tpu as pltpu
from jax.experimental.pallas import tpu_sc as plsc

D = 256
HID = 256
N_NODES = 10000
N_EDGES = 160000

NC = 2
NS = 16
NW = NC * NS

CHUNK = 128
SEG = 163840
TOTAL = 2 * SEG
CHUNKS_TOTAL = TOTAL // CHUNK
CHUNKS_PER_W = CHUNKS_TOTAL // NW

EDGE_BLK = 1280
NODE_BLK = 1000


def _proj_body(n_ref, w_ref, out_ref):
    out_ref[...] = jnp.dot(n_ref[...], w_ref[0],
                           preferred_element_type=jnp.float32)


def _node_projections(nfeat, w1_sd):
    grid = (2, N_NODES // NODE_BLK)
    return pl.pallas_call(
        _proj_body,
        grid=grid,
        in_specs=[
            pl.BlockSpec((NODE_BLK, D), lambda s, j: (j, 0)),
            pl.BlockSpec((1, D, HID), lambda s, j: (s, 0, 0)),
        ],
        out_specs=pl.BlockSpec((NODE_BLK, HID),
                               lambda s, j: (s * (N_NODES // NODE_BLK) + j, 0)),
        out_shape=jax.ShapeDtypeStruct((2 * N_NODES, HID), jnp.float32),
    )(nfeat, w1_sd)


@functools.cache
def _make_sc_gather():
    @functools.partial(
        pl.kernel,
        mesh=plsc.VectorSubcoreMesh(core_axis_name="c", subcore_axis_name="s"),
        out_type=jax.ShapeDtypeStruct((TOTAL, HID), jnp.float32),
        scratch_types=[
            pltpu.VMEM((CHUNKS_PER_W, CHUNK), jnp.int32),
            pltpu.VMEM((CHUNK, HID), jnp.float32),
            pltpu.SemaphoreType.DMA,
        ],
    )
    def _sc_gather(table_hbm, idx_hbm, out_hbm, idx_v, rows_v, sem):
        wid = lax.axis_index("s") * NC + lax.axis_index("c")
        base = wid * CHUNKS_PER_W
        pltpu.sync_copy(idx_hbm.at[pl.ds(base, CHUNKS_PER_W)], idx_v)

        def body(j, carry):
            pltpu.async_copy(table_hbm.at[idx_v.at[j]], rows_v, sem).wait()
            pltpu.sync_copy(rows_v,
                            out_hbm.at[pl.ds((base + j) * CHUNK, CHUNK)])
            return carry

        lax.fori_loop(0, CHUNKS_PER_W, body, 0)

    return _sc_gather


def _edge_body(e_ref, gs_ref, gd_ref, w1e_ref, w2_ref, b1_ref, b2_ref,
               sc_ref, bi_ref, out_ref):
    x = e_ref[...]
    h1 = jnp.dot(x, w1e_ref[...], preferred_element_type=jnp.float32)
    h1 = h1 + gs_ref[...] + gd_ref[...] + b1_ref[...]
    h1 = h1 * jax.nn.sigmoid(h1)
    h = jnp.dot(h1, w2_ref[...], preferred_element_type=jnp.float32)
    h = h + b2_ref[...]
    mean = jnp.mean(h, axis=-1, keepdims=True)
    c = h - mean
    var = jnp.mean(c * c, axis=-1, keepdims=True)
    out_ref[...] = c * lax.rsqrt(var + 1e-5) * sc_ref[...] + bi_ref[...] + x


def _edge_mlp(efeat, gathered, w1e, w2, b1, b2, ln_scale, ln_bias):
    grid = (N_EDGES // EDGE_BLK,)
    full = lambda i: (0, 0)
    return pl.pallas_call(
        _edge_body,
        grid=grid,
        in_specs=[
            pl.BlockSpec((EDGE_BLK, D), lambda i: (i, 0)),
            pl.BlockSpec((EDGE_BLK, HID), lambda i: (i, 0)),
            pl.BlockSpec((EDGE_BLK, HID), lambda i: (i + SEG // EDGE_BLK, 0)),
            pl.BlockSpec((D, HID), full),
            pl.BlockSpec((HID, D), full),
            pl.BlockSpec((1, HID), full),
            pl.BlockSpec((1, D), full),
            pl.BlockSpec((1, D), full),
            pl.BlockSpec((1, D), full),
        ],
        out_specs=pl.BlockSpec((EDGE_BLK, D), lambda i: (i, 0)),
        out_shape=jax.ShapeDtypeStruct((N_EDGES, D), jnp.float32),
    )(efeat, gathered, gathered, w1e, w2, b1, b2, ln_scale, ln_bias)


def kernel(efeat, nfeat, edge_index, w1, b1, w2, b2, ln_scale, ln_bias):
    src = edge_index[0].astype(jnp.int32)
    dst = edge_index[1].astype(jnp.int32)

    w1e = w1[:D]
    w1_sd = jnp.stack([w1[D:2 * D], w1[2 * D:]])

    table = _node_projections(nfeat, w1_sd)

    idx = jnp.zeros((TOTAL,), jnp.int32)
    idx = idx.at[:N_EDGES].set(src)
    idx = idx.at[SEG:SEG + N_EDGES].set(dst + N_NODES)
    idx2d = idx.reshape(CHUNKS_TOTAL, CHUNK)

    gathered = _make_sc_gather()(table, idx2d)

    out = _edge_mlp(efeat, gathered, w1e, w2,
                    b1.reshape(1, HID), b2.reshape(1, D),
                    ln_scale.reshape(1, D), ln_bias.reshape(1, D))
    return (out, nfeat)


# speedup vs baseline: 1.6989x; 1.0816x over previous
import functools

import jax
import jax.numpy as jnp
from jax import lax
from jax.experimental import pallas as pl
from jax.experimental.pallas import tpu as pltpu
from jax.experimental.pallas import tpu_sc as plsc

D = 256
HID = 256
N_NODES = 10000
N_EDGES = 160000

NC = 2
NS = 16
NW = NC * NS

CHUNK = 128
SEG = 163840
TOTAL = 2 * SEG
CHUNKS_TOTAL = TOTAL // CHUNK
CHUNKS_PER_W = CHUNKS_TOTAL // NW

EDGE_BLK = 1280
NODE_BLK = 1000


def _proj_body(n_ref, w_ref, out_ref):
    out_ref[...] = jnp.dot(n_ref[...], w_ref[0],
                           preferred_element_type=jnp.float32)


def _node_projections(nfeat, w1_sd):
    grid = (2, N_NODES // NODE_BLK)
    return pl.pallas_call(
        _proj_body,
        grid=grid,
        in_specs=[
            pl.BlockSpec((NODE_BLK, D), lambda s, j: (j, 0)),
            pl.BlockSpec((1, D, HID), lambda s, j: (s, 0, 0)),
        ],
        out_specs=pl.BlockSpec((NODE_BLK, HID),
                               lambda s, j: (s * (N_NODES // NODE_BLK) + j, 0)),
        out_shape=jax.ShapeDtypeStruct((2 * N_NODES, HID), jnp.float32),
    )(nfeat, w1_sd)


@functools.cache
def _make_sc_gather():
    @functools.partial(
        pl.kernel,
        mesh=plsc.VectorSubcoreMesh(core_axis_name="c", subcore_axis_name="s"),
        out_type=jax.ShapeDtypeStruct((TOTAL, HID), jnp.float32),
        scratch_types=[
            pltpu.VMEM((CHUNKS_PER_W, CHUNK), jnp.int32),
            pltpu.VMEM((CHUNK, HID), jnp.float32),
            pltpu.VMEM((CHUNK, HID), jnp.float32),
            pltpu.SemaphoreType.DMA,
            pltpu.SemaphoreType.DMA,
            pltpu.SemaphoreType.DMA,
            pltpu.SemaphoreType.DMA,
        ],
    )
    def _sc_gather(table_hbm, idx_hbm, out_hbm, idx_v,
                   rows0, rows1, g0, g1, s0, s1):
        wid = lax.axis_index("s") * NC + lax.axis_index("c")
        base = wid * CHUNKS_PER_W
        pltpu.sync_copy(idx_hbm.at[pl.ds(base, CHUNKS_PER_W)], idx_v)

        bufs = (rows0, rows1)
        gsems = (g0, g1)
        ssems = (s0, s1)

        pltpu.async_copy(table_hbm.at[idx_v.at[0]], rows0, g0)
        pltpu.async_copy(table_hbm.at[idx_v.at[1]], rows1, g1)

        def body(jo, carry):
            for b in range(2):
                c = jo * 2 + b
                buf = bufs[b]
                dst = out_hbm.at[pl.ds((base + c) * CHUNK, CHUNK)]
                pltpu.make_async_copy(table_hbm.at[idx_v.at[c]], buf,
                                      gsems[b]).wait()
                pltpu.async_copy(buf, dst, ssems[b])
                pltpu.make_async_copy(buf, dst, ssems[b]).wait()
                nxt = jnp.minimum(c + 2, CHUNKS_PER_W - 1)

                @pl.when(c + 2 < CHUNKS_PER_W)
                def _():
                    pltpu.async_copy(table_hbm.at[idx_v.at[nxt]], buf,
                                     gsems[b])
            return carry

        lax.fori_loop(0, CHUNKS_PER_W // 2, body, 0)

    return _sc_gather


def _edge_body(e_ref, gs_ref, gd_ref, w1e_ref, w2_ref, b1_ref, b2_ref,
               sc_ref, bi_ref, out_ref):
    x = e_ref[...]
    h1 = jnp.dot(x, w1e_ref[...], preferred_element_type=jnp.float32)
    h1 = h1 + gs_ref[...] + gd_ref[...] + b1_ref[...]
    h1 = h1 * jax.nn.sigmoid(h1)
    h = jnp.dot(h1, w2_ref[...], preferred_element_type=jnp.float32)
    h = h + b2_ref[...]
    mean = jnp.mean(h, axis=-1, keepdims=True)
    c = h - mean
    var = jnp.mean(c * c, axis=-1, keepdims=True)
    out_ref[...] = c * lax.rsqrt(var + 1e-5) * sc_ref[...] + bi_ref[...] + x


def _edge_mlp(efeat, gathered, w1e, w2, b1, b2, ln_scale, ln_bias):
    grid = (N_EDGES // EDGE_BLK,)
    full = lambda i: (0, 0)
    return pl.pallas_call(
        _edge_body,
        grid=grid,
        in_specs=[
            pl.BlockSpec((EDGE_BLK, D), lambda i: (i, 0)),
            pl.BlockSpec((EDGE_BLK, HID), lambda i: (i, 0)),
            pl.BlockSpec((EDGE_BLK, HID), lambda i: (i + SEG // EDGE_BLK, 0)),
            pl.BlockSpec((D, HID), full),
            pl.BlockSpec((HID, D), full),
            pl.BlockSpec((1, HID), full),
            pl.BlockSpec((1, D), full),
            pl.BlockSpec((1, D), full),
            pl.BlockSpec((1, D), full),
        ],
        out_specs=pl.BlockSpec((EDGE_BLK, D), lambda i: (i, 0)),
        out_shape=jax.ShapeDtypeStruct((N_EDGES, D), jnp.float32),
    )(efeat, gathered, gathered, w1e, w2, b1, b2, ln_scale, ln_bias)


def kernel(efeat, nfeat, edge_index, w1, b1, w2, b2, ln_scale, ln_bias):
    src = edge_index[0].astype(jnp.int32)
    dst = edge_index[1].astype(jnp.int32)

    w1e = w1[:D]
    w1_sd = jnp.stack([w1[D:2 * D], w1[2 * D:]])

    table = _node_projections(nfeat, w1_sd)

    idx = jnp.zeros((TOTAL,), jnp.int32)
    idx = idx.at[:N_EDGES].set(src)
    idx = idx.at[SEG:SEG + N_EDGES].set(dst + N_NODES)
    idx2d = idx.reshape(CHUNKS_TOTAL, CHUNK)

    gathered = _make_sc_gather()(table, idx2d)

    out = _edge_mlp(efeat, gathered, w1e, w2,
                    b1.reshape(1, HID), b2.reshape(1, D),
                    ln_scale.reshape(1, D), ln_bias.reshape(1, D))
    return (out, nfeat)
